# baseline (speedup 1.0000x reference)
import jax
import jax.numpy as jnp
from jax.experimental import pallas as pl


def _bn(h, g, b):
    mu = jnp.mean(h, axis=0)
    var = jnp.var(h, axis=0)
    return g * (h - mu) / jnp.sqrt(var + 1e-5) + b


def _genconv(h, src, dst, edge_emb, W, b, n):
    m = jax.nn.relu(jnp.take(h, src, axis=0) + edge_emb) + 1e-7
    mmax = jax.ops.segment_max(m, dst, num_segments=n)
    num = jnp.exp(m - jnp.take(mmax, dst, axis=0))
    den = jax.ops.segment_sum(num, dst, num_segments=n)
    alpha = num / (jnp.take(den, dst, axis=0) + 1e-16)
    agg = jax.ops.segment_sum(alpha * m, dst, num_segments=n)
    return (h + agg) @ W + b


def _copy_kernel(x_ref, o_ref):
    o_ref[...] = x_ref[...]


def kernel(x, edge_index, edge_attr, batch, atom_table, bond_table, Wg, bg, gamma, beta, W_out, b_out):
    n = x.shape[0]
    L = Wg.shape[0]
    B = 64
    src = edge_index[0]
    dst = edge_index[1]
    h = jnp.sum(jnp.take(atom_table, x + (jnp.arange(9) * 32)[None, :], axis=0), axis=1)
    e = jnp.sum(jnp.take(bond_table, edge_attr + (jnp.arange(3) * 8)[None, :], axis=0), axis=1)
    h = _genconv(h, src, dst, e, Wg[0], bg[0], n)
    for l in range(1, L):
        h2 = jax.nn.relu(_bn(h, gamma[l - 1], beta[l - 1]))
        h = _genconv(h2, src, dst, e, Wg[l], bg[l], n) + h
    h = _bn(h, gamma[L - 1], beta[L - 1])
    hg = jax.ops.segment_sum(h, batch, num_segments=B)
    out = hg @ W_out + b_out
    return pl.pallas_call(
        _copy_kernel,
        out_shape=jax.ShapeDtypeStruct(out.shape, out.dtype),
    )(out)


# speedup vs baseline: 6.0391x; 6.0391x over previous
import functools

import jax
import jax.numpy as jnp
from jax import lax
from jax.experimental import pallas as pl
from jax.experimental.pallas import tpu as pltpu
from jax.experimental.pallas import tpu_sc as plsc

_N = 10000
_E = 320000
_HID = 128
_NGRAPH = 64
_NLAYER = 7

_NTILE = 32
_NPT = 320
_NPAD = _NTILE * _NPT
_K = 128
_OFFW = 336

_BLKN = 1000
_BLKE = 1000

_f32 = jnp.float32


def _edge_body(h2_hbm, e_hbm, srcp_hbm, offw_hbm, agg_hbm,
               off_tv, idx_v, hsrc_v, e_v, aggout):
    c = lax.axis_index("c")
    s = lax.axis_index("s")
    wid = s * 2 + c
    n0 = wid * _NPT
    cnt = jnp.minimum(_NPT, _N - n0)

    pltpu.sync_copy(offw_hbm.at[pl.ds(wid * _OFFW, _OFFW)],
                    off_tv.at[pl.ds(0, _OFFW)])

    def _off2(j):
        v = off_tv[pl.ds(j, 16)]
        return v[0], v[1]

    e0, _ = _off2(0)
    e1, _ = _off2(cnt)

    zeros = jnp.zeros((16,), _f32)

    def _fetch(g):
        base = g * _K
        pltpu.sync_copy(srcp_hbm.at[pl.ds(base, _K)], idx_v)
        pltpu.sync_copy(h2_hbm.at[idx_v], hsrc_v)
        pltpu.sync_copy(e_hbm.at[pl.ds(base, _K)], e_v)

    @pl.when(e1 > e0)
    def _():
        _fetch(e0 // _K)

    def _node(ld, pos0):
        oa, ob = _off2(ld)
        r = ob - oa

        def _edge(_, st):
            pos = st[0]
            M = list(st[1:9])
            D = list(st[9:17])
            S = list(st[17:25])

            @pl.when(jnp.logical_and(jnp.bitwise_and(pos, _K - 1) == 0,
                                     pos > e0))
            def _():
                _fetch(pos // _K)

            i = jnp.bitwise_and(pos, _K - 1)
            for cc in range(8):
                hv = hsrc_v[i, pl.ds(cc * 16, 16)]
                ev = e_v[i, pl.ds(cc * 16, 16)]
                m = jnp.maximum(hv + ev, 0.0) + 1e-7
                mn = jnp.maximum(M[cc], m)
                a = jnp.exp(M[cc] - mn)
                b = jnp.exp(m - mn)
                D[cc] = D[cc] * a + b
                S[cc] = S[cc] * a + b * m
                M[cc] = mn
            return tuple([pos + 1] + M + D + S)

        st = lax.fori_loop(0, r, _edge, tuple([pos0] + [zeros] * 24))
        for cc in range(8):
            aggout[ld, pl.ds(cc * 16, 16)] = st[17 + cc] / (st[9 + cc] + 1e-16)
        return st[0]

    lax.fori_loop(0, cnt, _node, e0)

    pltpu.sync_copy(aggout, agg_hbm.at[pl.ds(n0, _NPT)])


_edge_call = pl.kernel(
    _edge_body,
    out_type=jax.ShapeDtypeStruct((_NPAD, _HID), _f32),
    mesh=plsc.VectorSubcoreMesh(core_axis_name="c", subcore_axis_name="s",
                                num_cores=2, num_subcores=16),
    compiler_params=pltpu.CompilerParams(needs_layout_passes=False),
    scratch_types=[
        pltpu.VMEM((_OFFW + 16,), jnp.int32),
        pltpu.VMEM((_K,), jnp.int32),
        pltpu.VMEM((_K, _HID), _f32),
        pltpu.VMEM((_K, _HID), _f32),
        pltpu.VMEM((_NPT, _HID), _f32),
    ],
)


def _enc_body(ncat, ncols, xb_ref, tab_ref, out_ref):
    xb = xb_ref[...]
    blk = xb.shape[0]
    mh = jnp.zeros((blk, ncols), _f32)
    for k in range(ncat):
        col = xb[:, k:k + 1] + jnp.int32(k * (ncols // ncat))
        mh = mh + (col == lax.broadcasted_iota(jnp.int32, (blk, ncols), 1)
                   ).astype(_f32)
    out_ref[...] = jnp.dot(mh, tab_ref[...], preferred_element_type=_f32,
                           precision=lax.Precision.HIGHEST)


def _make_enc(nrows, blk, ncat, ncols):
    return pl.pallas_call(
        functools.partial(_enc_body, ncat, ncols),
        grid=(nrows // blk,),
        in_specs=[
            pl.BlockSpec((blk, ncat), lambda i: (i, 0)),
            pl.BlockSpec((ncols, _HID), lambda i: (0, 0)),
        ],
        out_specs=pl.BlockSpec((blk, _HID), lambda i: (i, 0)),
        out_shape=jax.ShapeDtypeStruct((nrows, _HID), _f32),
    )


_atom_call = _make_enc(_N, _BLKN, 9, 288)
_bond_call = _make_enc(_E, _BLKE, 3, 24)


def _upd_body(res, h2_ref, h_ref, agg_ref, w_ref, b_ref,
              hnew_ref, s1_ref, stat):
    i = pl.program_id(0)

    @pl.when(i == 0)
    def _():
        stat[...] = jnp.zeros_like(stat)

    hn = jnp.dot(h2_ref[...] + agg_ref[...], w_ref[...],
                 preferred_element_type=_f32,
                 precision=lax.Precision.HIGHEST) + b_ref[...]
    if res:
        hn = hn + h_ref[...]
    hnew_ref[...] = hn
    stat[0:1, :] += jnp.sum(hn, axis=0, keepdims=True)

    @pl.when(i == pl.num_programs(0) - 1)
    def _():
        s1_ref[...] = stat[...]


def _make_upd(res):
    return pl.pallas_call(
        functools.partial(_upd_body, res),
        grid=(_N // _BLKN,),
        in_specs=[
            pl.BlockSpec((_BLKN, _HID), lambda i: (i, 0)),
            pl.BlockSpec((_BLKN, _HID), lambda i: (i, 0)),
            pl.BlockSpec((_BLKN, _HID), lambda i: (i, 0)),
            pl.BlockSpec((_HID, _HID), lambda i: (0, 0)),
            pl.BlockSpec((1, _HID), lambda i: (0, 0)),
        ],
        out_specs=[
            pl.BlockSpec((_BLKN, _HID), lambda i: (i, 0)),
            pl.BlockSpec((1, _HID), lambda i: (0, 0)),
        ],
        out_shape=[
            jax.ShapeDtypeStruct((_N, _HID), _f32),
            jax.ShapeDtypeStruct((1, _HID), _f32),
        ],
        scratch_shapes=[pltpu.VMEM((1, _HID), _f32)],
    )


_upd0_call = _make_upd(False)
_upd_call = _make_upd(True)


def _stats_body(h_ref, s1_ref, g_ref, be_ref, par_ref, acc):
    i = pl.program_id(0)

    @pl.when(i == 0)
    def _():
        acc[...] = jnp.zeros_like(acc)

    mu = s1_ref[...] * (1.0 / _N)
    d = h_ref[...] - mu
    acc[...] += jnp.sum(d * d, axis=0, keepdims=True)

    @pl.when(i == pl.num_programs(0) - 1)
    def _():
        var = acc[...] * (1.0 / _N)
        sc = g_ref[...] * lax.rsqrt(var + 1e-5)
        par_ref[0:1, :] = sc
        par_ref[1:2, :] = be_ref[...] - mu * sc


_stats_call = pl.pallas_call(
    _stats_body,
    grid=(_N // _BLKN,),
    in_specs=[
        pl.BlockSpec((_BLKN, _HID), lambda i: (i, 0)),
        pl.BlockSpec((1, _HID), lambda i: (0, 0)),
        pl.BlockSpec((1, _HID), lambda i: (0, 0)),
        pl.BlockSpec((1, _HID), lambda i: (0, 0)),
    ],
    out_specs=pl.BlockSpec((2, _HID), lambda i: (0, 0)),
    out_shape=jax.ShapeDtypeStruct((2, _HID), _f32),
    scratch_shapes=[pltpu.VMEM((1, _HID), _f32)],
)


def _act_body(h_ref, p_ref, out_ref):
    out_ref[...] = jnp.maximum(h_ref[...] * p_ref[0:1, :] + p_ref[1:2, :], 0.0)


_act_call = pl.pallas_call(
    _act_body,
    grid=(_N // _BLKN,),
    in_specs=[
        pl.BlockSpec((_BLKN, _HID), lambda i: (i, 0)),
        pl.BlockSpec((2, _HID), lambda i: (0, 0)),
    ],
    out_specs=pl.BlockSpec((_BLKN, _HID), lambda i: (i, 0)),
    out_shape=jax.ShapeDtypeStruct((_N, _HID), _f32),
)


def _pool_body(h_ref, p_ref, b2_ref, wo_ref, bo_ref, out_ref, acc):
    i = pl.program_id(0)

    @pl.when(i == 0)
    def _():
        acc[...] = jnp.zeros_like(acc)

    hb = h_ref[...] * p_ref[0:1, :] + p_ref[1:2, :]
    oh = (b2_ref[...] == lax.broadcasted_iota(jnp.int32, (_BLKN, _NGRAPH), 1)
          ).astype(_f32)
    acc[...] += lax.dot_general(oh, hb, (((0,), (0,)), ((), ())),
                                preferred_element_type=_f32,
                                precision=lax.Precision.HIGHEST)

    @pl.when(i == pl.num_programs(0) - 1)
    def _():
        out_ref[...] = jnp.dot(acc[...], wo_ref[...],
                               preferred_element_type=_f32,
                               precision=lax.Precision.HIGHEST) + bo_ref[...]


_pool_call = pl.pallas_call(
    _pool_body,
    grid=(_N // _BLKN,),
    in_specs=[
        pl.BlockSpec((_BLKN, _HID), lambda i: (i, 0)),
        pl.BlockSpec((2, _HID), lambda i: (0, 0)),
        pl.BlockSpec((_BLKN, 1), lambda i: (i, 0)),
        pl.BlockSpec((_HID, 10), lambda i: (0, 0)),
        pl.BlockSpec((1, 10), lambda i: (0, 0)),
    ],
    out_specs=pl.BlockSpec((_NGRAPH, 10), lambda i: (0, 0)),
    out_shape=jax.ShapeDtypeStruct((_NGRAPH, 10), _f32),
    scratch_shapes=[pltpu.VMEM((_NGRAPH, _HID), _f32)],
)


def kernel(x, edge_index, edge_attr, batch, atom_table, bond_table, Wg, bg,
           gamma, beta, W_out, b_out):
    src = edge_index[0]
    dst = edge_index[1]

    perm = jnp.argsort(dst)
    dst_s = jnp.take(dst, perm)
    src_p = jnp.take(src, perm).astype(jnp.int32)
    ea_s = jnp.take(edge_attr, perm, axis=0).astype(jnp.int32)
    offsets = jnp.searchsorted(
        dst_s, jnp.arange(_N + 1, dtype=jnp.int32)).astype(jnp.int32)
    npad_off = (_NTILE - 1) * _NPT + _OFFW
    offp = jnp.concatenate(
        [offsets, jnp.full((npad_off - (_N + 1),), _E, jnp.int32)])
    offw = offp[(jnp.arange(_NTILE, dtype=jnp.int32) * _NPT)[:, None]
                + jnp.arange(_OFFW, dtype=jnp.int32)[None, :]].reshape(-1)

    h0 = _atom_call(x.astype(jnp.int32), atom_table)
    e_s = _bond_call(ea_s, bond_table)

    def edge_pass(h2):
        return _edge_call(h2, e_s, src_p, offw)[:_N]

    agg = edge_pass(h0)
    h, s1 = _upd0_call(h0, h0, agg, Wg[0], bg[0:1])
    par = _stats_call(h, s1, gamma[0:1], beta[0:1])
    for l in range(1, _NLAYER):
        h2 = _act_call(h, par)
        agg = edge_pass(h2)
        h, s1 = _upd_call(h2, h, agg, Wg[l], bg[l:l + 1])
        par = _stats_call(h, s1, gamma[l:l + 1], beta[l:l + 1])
    out = _pool_call(h, par, batch.reshape(_N, 1).astype(jnp.int32),
                     W_out, b_out.reshape(1, 10))
    return out


# speedup vs baseline: 6.4811x; 1.0732x over previous
import functools

import jax
import jax.numpy as jnp
from jax import lax
from jax.experimental import pallas as pl
from jax.experimental.pallas import tpu as pltpu
from jax.experimental.pallas import tpu_sc as plsc

_N = 10000
_E = 320000
_HID = 128
_NGRAPH = 64
_NLAYER = 7

_NTILE = 32
_NPT = 320
_NPAD = _NTILE * _NPT
_K = 128
_OFFW = 336

_BLKN = 1000
_BLKE = 1000

_f32 = jnp.float32


_SB = 8
_NBLK = _E // _K
_NBLKP = _NBLK + 2 * _SB


def _edge_body(h2_hbm, e_hbm, srcp_hbm, offw_hbm, agg_hbm,
               off_tv, idx_sb, hsrc_v, e_v, aggout, sem_d, sem_i):
    c = lax.axis_index("c")
    s = lax.axis_index("s")
    wid = s * 2 + c
    n0 = wid * _NPT
    cnt = jnp.minimum(_NPT, _N - n0)

    pltpu.sync_copy(offw_hbm.at[pl.ds(wid * _OFFW, _OFFW)],
                    off_tv.at[pl.ds(0, _OFFW)])

    def _off2(j):
        v = off_tv[pl.ds(j, 16)]
        return v[0], v[1]

    e0, _ = _off2(0)
    e1, _ = _off2(cnt)

    zeros = jnp.zeros((16,), _f32)
    has = e1 > e0

    gd0 = e0 // _K
    gend = jnp.where(has, (e1 - 1) // _K + 1, gd0)
    nbt = gend - gd0
    qb0 = gd0 // _SB
    last_q = (gend - 1) // _SB - qb0

    def _issue_idx(q):
        pltpu.async_copy(srcp_hbm.at[pl.ds((qb0 + q) * _SB, _SB)],
                         idx_sb.at[lax.rem(q, 2)], sem_i.at[lax.rem(q, 2)])

    def _wait_idx(q):
        pltpu.make_async_copy(srcp_hbm.at[pl.ds((qb0 + q) * _SB, _SB)],
                              idx_sb.at[lax.rem(q, 2)],
                              sem_i.at[lax.rem(q, 2)]).wait()

    def _chunk_slot_row(t):
        g = gd0 + t
        return lax.rem(g // _SB - qb0, 2), lax.rem(g, _SB)

    def _issue_data(t):
        slot = lax.rem(t, 2)
        cs, row = _chunk_slot_row(t)
        pltpu.async_copy(h2_hbm.at[idx_sb.at[cs, row]],
                         hsrc_v.at[slot], sem_d.at[slot])
        pltpu.async_copy(e_hbm.at[pl.ds((gd0 + t) * _K, _K)], e_v.at[slot],
                         sem_d.at[slot])

    def _wait_data(t):
        slot = lax.rem(t, 2)
        cs, row = _chunk_slot_row(t)
        pltpu.make_async_copy(h2_hbm.at[idx_sb.at[cs, row]],
                              hsrc_v.at[slot], sem_d.at[slot]).wait()
        pltpu.make_async_copy(e_hbm.at[pl.ds((gd0 + t) * _K, _K)],
                              e_v.at[slot], sem_d.at[slot]).wait()

    @pl.when(has)
    def _():
        _issue_idx(0)
        _wait_idx(0)

        @pl.when(last_q >= 1)
        def _():
            _issue_idx(1)

        _issue_data(0)

        @pl.when(nbt >= 2)
        def _():
            @pl.when(lax.rem(gd0 + 1, _SB) == 0)
            def _():
                _wait_idx(1)

                @pl.when(last_q >= 2)
                def _():
                    _issue_idx(2)

            _issue_data(1)

        _wait_data(0)

    def _node(ld, st0):
        oa, ob = _off2(ld)
        r = ob - oa

        def _edge(_, st):
            pos = st[0]
            M = list(st[1:9])
            D = list(st[9:17])
            S = list(st[17:25])
            t = pos // _K - gd0

            @pl.when(jnp.logical_and(jnp.bitwise_and(pos, _K - 1) == 0,
                                     pos > e0))
            def _():
                tn = t + 1
                cn = (gd0 + tn) // _SB - qb0

                @pl.when(jnp.logical_and(lax.rem(gd0 + tn, _SB) == 0,
                                         cn <= last_q))
                def _():
                    _wait_idx(cn)

                    @pl.when(cn + 1 <= last_q)
                    def _():
                        _issue_idx(cn + 1)

                @pl.when(tn < nbt)
                def _():
                    _issue_data(tn)

                _wait_data(t)

            slot = lax.rem(t, 2)
            i = jnp.bitwise_and(pos, _K - 1)
            for cc in range(8):
                hv = hsrc_v[slot, i, pl.ds(cc * 16, 16)]
                ev = e_v[slot, i, pl.ds(cc * 16, 16)]
                m = jnp.maximum(hv + ev, 0.0) + 1e-7
                mn = jnp.maximum(M[cc], m)
                a = jnp.exp(M[cc] - mn)
                b = jnp.exp(m - mn)
                D[cc] = D[cc] * a + b
                S[cc] = S[cc] * a + b * m
                M[cc] = mn
            return tuple([pos + 1] + M + D + S)

        st = lax.fori_loop(0, r, _edge, tuple([st0] + [zeros] * 24))
        for cc in range(8):
            aggout[ld, pl.ds(cc * 16, 16)] = st[17 + cc] / (st[9 + cc] + 1e-16)
        return st[0]

    lax.fori_loop(0, cnt, _node, e0)

    pltpu.sync_copy(aggout, agg_hbm.at[pl.ds(n0, _NPT)])


_edge_call = pl.kernel(
    _edge_body,
    out_type=jax.ShapeDtypeStruct((_NPAD, _HID), _f32),
    mesh=plsc.VectorSubcoreMesh(core_axis_name="c", subcore_axis_name="s",
                                num_cores=2, num_subcores=16),
    compiler_params=pltpu.CompilerParams(needs_layout_passes=False),
    scratch_types=[
        pltpu.VMEM((_OFFW + 16,), jnp.int32),
        pltpu.VMEM((2, _SB, _K), jnp.int32),
        pltpu.VMEM((2, _K, _HID), _f32),
        pltpu.VMEM((2, _K, _HID), _f32),
        pltpu.VMEM((_NPT, _HID), _f32),
        pltpu.SemaphoreType.DMA((2,)),
        pltpu.SemaphoreType.DMA((2,)),
    ],
)


def _enc_body(ncat, ncols, xb_ref, tab_ref, out_ref):
    xb = xb_ref[...]
    blk = xb.shape[0]
    mh = jnp.zeros((blk, ncols), _f32)
    for k in range(ncat):
        col = xb[:, k:k + 1] + jnp.int32(k * (ncols // ncat))
        mh = mh + (col == lax.broadcasted_iota(jnp.int32, (blk, ncols), 1)
                   ).astype(_f32)
    out_ref[...] = jnp.dot(mh, tab_ref[...], preferred_element_type=_f32,
                           precision=lax.Precision.HIGHEST)


def _make_enc(nrows, blk, ncat, ncols):
    return pl.pallas_call(
        functools.partial(_enc_body, ncat, ncols),
        grid=(nrows // blk,),
        in_specs=[
            pl.BlockSpec((blk, ncat), lambda i: (i, 0)),
            pl.BlockSpec((ncols, _HID), lambda i: (0, 0)),
        ],
        out_specs=pl.BlockSpec((blk, _HID), lambda i: (i, 0)),
        out_shape=jax.ShapeDtypeStruct((nrows, _HID), _f32),
    )


_atom_call = _make_enc(_N, _BLKN, 9, 288)
_bond_call = _make_enc(_E, _BLKE, 3, 24)


def _upd_body(res, h2_ref, h_ref, agg_ref, w_ref, b_ref,
              hnew_ref, s1_ref, stat):
    i = pl.program_id(0)

    @pl.when(i == 0)
    def _():
        stat[...] = jnp.zeros_like(stat)

    hn = jnp.dot(h2_ref[...] + agg_ref[...], w_ref[...],
                 preferred_element_type=_f32,
                 precision=lax.Precision.HIGHEST) + b_ref[...]
    if res:
        hn = hn + h_ref[...]
    hnew_ref[...] = hn
    stat[0:1, :] += jnp.sum(hn, axis=0, keepdims=True)

    @pl.when(i == pl.num_programs(0) - 1)
    def _():
        s1_ref[...] = stat[...]


def _make_upd(res):
    return pl.pallas_call(
        functools.partial(_upd_body, res),
        grid=(_N // _BLKN,),
        in_specs=[
            pl.BlockSpec((_BLKN, _HID), lambda i: (i, 0)),
            pl.BlockSpec((_BLKN, _HID), lambda i: (i, 0)),
            pl.BlockSpec((_BLKN, _HID), lambda i: (i, 0)),
            pl.BlockSpec((_HID, _HID), lambda i: (0, 0)),
            pl.BlockSpec((1, _HID), lambda i: (0, 0)),
        ],
        out_specs=[
            pl.BlockSpec((_BLKN, _HID), lambda i: (i, 0)),
            pl.BlockSpec((1, _HID), lambda i: (0, 0)),
        ],
        out_shape=[
            jax.ShapeDtypeStruct((_N, _HID), _f32),
            jax.ShapeDtypeStruct((1, _HID), _f32),
        ],
        scratch_shapes=[pltpu.VMEM((1, _HID), _f32)],
    )


_upd0_call = _make_upd(False)
_upd_call = _make_upd(True)


def _stats_body(h_ref, s1_ref, g_ref, be_ref, par_ref, acc):
    i = pl.program_id(0)

    @pl.when(i == 0)
    def _():
        acc[...] = jnp.zeros_like(acc)

    mu = s1_ref[...] * (1.0 / _N)
    d = h_ref[...] - mu
    acc[...] += jnp.sum(d * d, axis=0, keepdims=True)

    @pl.when(i == pl.num_programs(0) - 1)
    def _():
        var = acc[...] * (1.0 / _N)
        sc = g_ref[...] / jnp.sqrt(var + 1e-5)
        par_ref[0:1, :] = sc
        par_ref[1:2, :] = be_ref[...] - mu * sc


_stats_call = pl.pallas_call(
    _stats_body,
    grid=(_N // _BLKN,),
    in_specs=[
        pl.BlockSpec((_BLKN, _HID), lambda i: (i, 0)),
        pl.BlockSpec((1, _HID), lambda i: (0, 0)),
        pl.BlockSpec((1, _HID), lambda i: (0, 0)),
        pl.BlockSpec((1, _HID), lambda i: (0, 0)),
    ],
    out_specs=pl.BlockSpec((2, _HID), lambda i: (0, 0)),
    out_shape=jax.ShapeDtypeStruct((2, _HID), _f32),
    scratch_shapes=[pltpu.VMEM((1, _HID), _f32)],
)


def _act_body(h_ref, p_ref, out_ref):
    out_ref[...] = jnp.maximum(h_ref[...] * p_ref[0:1, :] + p_ref[1:2, :], 0.0)


_act_call = pl.pallas_call(
    _act_body,
    grid=(_N // _BLKN,),
    in_specs=[
        pl.BlockSpec((_BLKN, _HID), lambda i: (i, 0)),
        pl.BlockSpec((2, _HID), lambda i: (0, 0)),
    ],
    out_specs=pl.BlockSpec((_BLKN, _HID), lambda i: (i, 0)),
    out_shape=jax.ShapeDtypeStruct((_N, _HID), _f32),
)


def _pool_body(h_ref, p_ref, b2_ref, wo_ref, bo_ref, out_ref, acc):
    i = pl.program_id(0)

    @pl.when(i == 0)
    def _():
        acc[...] = jnp.zeros_like(acc)

    hb = h_ref[...] * p_ref[0:1, :] + p_ref[1:2, :]
    oh = (b2_ref[...] == lax.broadcasted_iota(jnp.int32, (_BLKN, _NGRAPH), 1)
          ).astype(_f32)
    acc[...] += lax.dot_general(oh, hb, (((0,), (0,)), ((), ())),
                                preferred_element_type=_f32,
                                precision=lax.Precision.HIGHEST)

    @pl.when(i == pl.num_programs(0) - 1)
    def _():
        out_ref[...] = jnp.dot(acc[...], wo_ref[...],
                               preferred_element_type=_f32,
                               precision=lax.Precision.HIGHEST) + bo_ref[...]


_pool_call = pl.pallas_call(
    _pool_body,
    grid=(_N // _BLKN,),
    in_specs=[
        pl.BlockSpec((_BLKN, _HID), lambda i: (i, 0)),
        pl.BlockSpec((2, _HID), lambda i: (0, 0)),
        pl.BlockSpec((_BLKN, 1), lambda i: (i, 0)),
        pl.BlockSpec((_HID, 10), lambda i: (0, 0)),
        pl.BlockSpec((1, 10), lambda i: (0, 0)),
    ],
    out_specs=pl.BlockSpec((_NGRAPH, 10), lambda i: (0, 0)),
    out_shape=jax.ShapeDtypeStruct((_NGRAPH, 10), _f32),
    scratch_shapes=[pltpu.VMEM((_NGRAPH, _HID), _f32)],
)


def kernel(x, edge_index, edge_attr, batch, atom_table, bond_table, Wg, bg,
           gamma, beta, W_out, b_out):
    src = edge_index[0]
    dst = edge_index[1]

    perm = jnp.argsort(dst)
    dst_s = jnp.take(dst, perm)
    src_p = jnp.pad(jnp.take(src, perm).astype(jnp.int32),
                    (0, _NBLKP * _K - _E)).reshape(_NBLKP, _K)
    ea_s = jnp.take(edge_attr, perm, axis=0).astype(jnp.int32)
    offsets = jnp.searchsorted(
        dst_s, jnp.arange(_N + 1, dtype=jnp.int32)).astype(jnp.int32)
    npad_off = (_NTILE - 1) * _NPT + _OFFW
    offp = jnp.concatenate(
        [offsets, jnp.full((npad_off - (_N + 1),), _E, jnp.int32)])
    offw = offp[(jnp.arange(_NTILE, dtype=jnp.int32) * _NPT)[:, None]
                + jnp.arange(_OFFW, dtype=jnp.int32)[None, :]].reshape(-1)

    h0 = _atom_call(x.astype(jnp.int32), atom_table)
    e_s = _bond_call(ea_s, bond_table)

    def edge_pass(h2):
        return _edge_call(h2, e_s, src_p, offw)[:_N]

    agg = edge_pass(h0)
    h, s1 = _upd0_call(h0, h0, agg, Wg[0], bg[0:1])
    par = _stats_call(h, s1, gamma[0:1], beta[0:1])
    for l in range(1, _NLAYER):
        h2 = _act_call(h, par)
        agg = edge_pass(h2)
        h, s1 = _upd_call(h2, h, agg, Wg[l], bg[l:l + 1])
        par = _stats_call(h, s1, gamma[l:l + 1], beta[l:l + 1])
    out = _pool_call(h, par, batch.reshape(_N, 1).astype(jnp.int32),
                     W_out, b_out.reshape(1, 10))
    return out


# speedup vs baseline: 7.7923x; 1.2023x over previous
import functools

import jax
import jax.numpy as jnp
from jax import lax
from jax.experimental import pallas as pl
from jax.experimental.pallas import tpu as pltpu
from jax.experimental.pallas import tpu_sc as plsc

_N = 10000
_E = 320000
_HID = 128
_NGRAPH = 64
_NLAYER = 7

_NTILE = 32
_NPT = 320
_NPAD = _NTILE * _NPT
_K = 128
_OFFW = 336

_BLKN = 1000
_BLKE = 1000

_f32 = jnp.float32


_SB = 8
_NBLK = _E // _K
_NBLKP = _NBLK + 2 * _SB


def _edge_body(h2_hbm, e_hbm, srcp_hbm, offw_hbm, agg_hbm,
               off_tv, idx_sb, hsrc_v, e_v, aggout, sem_d, sem_i):
    c = lax.axis_index("c")
    s = lax.axis_index("s")
    wid = s * 2 + c
    n0 = wid * _NPT
    cnt = jnp.minimum(_NPT, _N - n0)

    pltpu.sync_copy(offw_hbm.at[pl.ds(wid * _OFFW, _OFFW)],
                    off_tv.at[pl.ds(0, _OFFW)])

    def _off2(j):
        v = off_tv[pl.ds(j, 16)]
        return v[0], v[1]

    e0, _ = _off2(0)
    e1, _ = _off2(cnt)

    zeros = jnp.zeros((16,), _f32)
    has = e1 > e0

    gd0 = e0 // _K
    gend = jnp.where(has, (e1 - 1) // _K + 1, gd0)
    nbt = gend - gd0
    qb0 = gd0 // _SB
    last_q = (gend - 1) // _SB - qb0

    def _issue_idx(q):
        pltpu.async_copy(srcp_hbm.at[pl.ds((qb0 + q) * _SB, _SB)],
                         idx_sb.at[lax.rem(q, 2)], sem_i.at[lax.rem(q, 2)])

    def _wait_idx(q):
        pltpu.make_async_copy(srcp_hbm.at[pl.ds((qb0 + q) * _SB, _SB)],
                              idx_sb.at[lax.rem(q, 2)],
                              sem_i.at[lax.rem(q, 2)]).wait()

    def _chunk_slot_row(t):
        g = gd0 + t
        return lax.rem(g // _SB - qb0, 2), lax.rem(g, _SB)

    def _issue_data(t):
        slot = lax.rem(t, 2)
        cs, row = _chunk_slot_row(t)
        pltpu.async_copy(h2_hbm.at[idx_sb.at[cs, row]],
                         hsrc_v.at[slot], sem_d.at[slot])
        pltpu.async_copy(e_hbm.at[pl.ds((gd0 + t) * _K, _K)], e_v.at[slot],
                         sem_d.at[slot])

    def _wait_data(t):
        slot = lax.rem(t, 2)
        cs, row = _chunk_slot_row(t)
        pltpu.make_async_copy(h2_hbm.at[idx_sb.at[cs, row]],
                              hsrc_v.at[slot], sem_d.at[slot]).wait()
        pltpu.make_async_copy(e_hbm.at[pl.ds((gd0 + t) * _K, _K)],
                              e_v.at[slot], sem_d.at[slot]).wait()

    @pl.when(has)
    def _():
        _issue_idx(0)
        _wait_idx(0)

        @pl.when(last_q >= 1)
        def _():
            _issue_idx(1)

        _issue_data(0)

        @pl.when(nbt >= 2)
        def _():
            @pl.when(lax.rem(gd0 + 1, _SB) == 0)
            def _():
                _wait_idx(1)

                @pl.when(last_q >= 2)
                def _():
                    _issue_idx(2)

            _issue_data(1)

        _wait_data(0)

    def _node(ld, st0):
        oa, ob = _off2(ld)
        r = ob - oa

        def _edge(_, st):
            pos = st[0]
            M = list(st[1:9])
            D = list(st[9:17])
            S = list(st[17:25])
            t = pos // _K - gd0

            @pl.when(jnp.logical_and(jnp.bitwise_and(pos, _K - 1) == 0,
                                     pos > e0))
            def _():
                tn = t + 1
                cn = (gd0 + tn) // _SB - qb0

                @pl.when(jnp.logical_and(lax.rem(gd0 + tn, _SB) == 0,
                                         cn <= last_q))
                def _():
                    _wait_idx(cn)

                    @pl.when(cn + 1 <= last_q)
                    def _():
                        _issue_idx(cn + 1)

                @pl.when(tn < nbt)
                def _():
                    _issue_data(tn)

                _wait_data(t)

            slot = lax.rem(t, 2)
            i = jnp.bitwise_and(pos, _K - 1)
            for cc in range(8):
                hv = hsrc_v[slot, i, pl.ds(cc * 16, 16)]
                ev = e_v[slot, i, pl.ds(cc * 16, 16)]
                m = jnp.maximum(hv + ev, 0.0) + 1e-7
                mn = jnp.maximum(M[cc], m)
                a = jnp.exp(M[cc] - mn)
                b = jnp.exp(m - mn)
                D[cc] = D[cc] * a + b
                S[cc] = S[cc] * a + b * m
                M[cc] = mn
            return tuple([pos + 1] + M + D + S)

        st = lax.fori_loop(0, r, _edge, tuple([st0] + [zeros] * 24))
        for cc in range(8):
            aggout[ld, pl.ds(cc * 16, 16)] = st[17 + cc] / (st[9 + cc] + 1e-16)
        return st[0]

    lax.fori_loop(0, cnt, _node, e0)

    pltpu.sync_copy(aggout, agg_hbm.at[pl.ds(n0, _NPT)])


_edge_call = pl.kernel(
    _edge_body,
    out_type=jax.ShapeDtypeStruct((_NPAD, _HID), _f32),
    mesh=plsc.VectorSubcoreMesh(core_axis_name="c", subcore_axis_name="s",
                                num_cores=2, num_subcores=16),
    compiler_params=pltpu.CompilerParams(needs_layout_passes=False),
    scratch_types=[
        pltpu.VMEM((_OFFW + 16,), jnp.int32),
        pltpu.VMEM((2, _SB, _K), jnp.int32),
        pltpu.VMEM((2, _K, _HID), _f32),
        pltpu.VMEM((2, _K, _HID), _f32),
        pltpu.VMEM((_NPT, _HID), _f32),
        pltpu.SemaphoreType.DMA((2,)),
        pltpu.SemaphoreType.DMA((2,)),
    ],
)


def _enc_body(ncat, ncols, xb_ref, tab_ref, out_ref):
    xb = xb_ref[...]
    blk = xb.shape[0]
    mh = jnp.zeros((blk, ncols), _f32)
    for k in range(ncat):
        col = xb[:, k:k + 1] + jnp.int32(k * (ncols // ncat))
        mh = mh + (col == lax.broadcasted_iota(jnp.int32, (blk, ncols), 1)
                   ).astype(_f32)
    out_ref[...] = jnp.dot(mh, tab_ref[...], preferred_element_type=_f32,
                           precision=lax.Precision.HIGHEST)


def _make_enc(nrows, blk, ncat, ncols):
    return pl.pallas_call(
        functools.partial(_enc_body, ncat, ncols),
        grid=(nrows // blk,),
        in_specs=[
            pl.BlockSpec((blk, ncat), lambda i: (i, 0)),
            pl.BlockSpec((ncols, _HID), lambda i: (0, 0)),
        ],
        out_specs=pl.BlockSpec((blk, _HID), lambda i: (i, 0)),
        out_shape=jax.ShapeDtypeStruct((nrows, _HID), _f32),
    )


_atom_call = _make_enc(_N, _BLKN, 9, 288)
_bond_call = _make_enc(_E, _BLKE, 3, 24)


def _upd_body(res, h2_ref, h_ref, agg_ref, w_ref, b_ref,
              hnew_ref, s1_ref, stat):
    i = pl.program_id(0)

    @pl.when(i == 0)
    def _():
        stat[...] = jnp.zeros_like(stat)

    hn = jnp.dot(h2_ref[...] + agg_ref[...], w_ref[...],
                 preferred_element_type=_f32,
                 precision=lax.Precision.HIGHEST) + b_ref[...]
    if res:
        hn = hn + h_ref[...]
    hnew_ref[...] = hn
    stat[0:1, :] += jnp.sum(hn, axis=0, keepdims=True)

    @pl.when(i == pl.num_programs(0) - 1)
    def _():
        s1_ref[...] = stat[...]


def _make_upd(res):
    return pl.pallas_call(
        functools.partial(_upd_body, res),
        grid=(_N // _BLKN,),
        in_specs=[
            pl.BlockSpec((_BLKN, _HID), lambda i: (i, 0)),
            pl.BlockSpec((_BLKN, _HID), lambda i: (i, 0)),
            pl.BlockSpec((_BLKN, _HID), lambda i: (i, 0)),
            pl.BlockSpec((_HID, _HID), lambda i: (0, 0)),
            pl.BlockSpec((1, _HID), lambda i: (0, 0)),
        ],
        out_specs=[
            pl.BlockSpec((_BLKN, _HID), lambda i: (i, 0)),
            pl.BlockSpec((1, _HID), lambda i: (0, 0)),
        ],
        out_shape=[
            jax.ShapeDtypeStruct((_N, _HID), _f32),
            jax.ShapeDtypeStruct((1, _HID), _f32),
        ],
        scratch_shapes=[pltpu.VMEM((1, _HID), _f32)],
    )


_upd0_call = _make_upd(False)
_upd_call = _make_upd(True)


def _stats_body(h_ref, s1_ref, g_ref, be_ref, par_ref, acc):
    i = pl.program_id(0)

    @pl.when(i == 0)
    def _():
        acc[...] = jnp.zeros_like(acc)

    mu = s1_ref[...] * (1.0 / _N)
    d = h_ref[...] - mu
    acc[...] += jnp.sum(d * d, axis=0, keepdims=True)

    @pl.when(i == pl.num_programs(0) - 1)
    def _():
        var = acc[...] * (1.0 / _N)
        sc = g_ref[...] / jnp.sqrt(var + 1e-5)
        par_ref[0:1, :] = sc
        par_ref[1:2, :] = be_ref[...] - mu * sc


_stats_call = pl.pallas_call(
    _stats_body,
    grid=(_N // _BLKN,),
    in_specs=[
        pl.BlockSpec((_BLKN, _HID), lambda i: (i, 0)),
        pl.BlockSpec((1, _HID), lambda i: (0, 0)),
        pl.BlockSpec((1, _HID), lambda i: (0, 0)),
        pl.BlockSpec((1, _HID), lambda i: (0, 0)),
    ],
    out_specs=pl.BlockSpec((2, _HID), lambda i: (0, 0)),
    out_shape=jax.ShapeDtypeStruct((2, _HID), _f32),
    scratch_shapes=[pltpu.VMEM((1, _HID), _f32)],
)


def _act_body(h_ref, p_ref, out_ref):
    out_ref[...] = jnp.maximum(h_ref[...] * p_ref[0:1, :] + p_ref[1:2, :], 0.0)


_act_call = pl.pallas_call(
    _act_body,
    grid=(_N // _BLKN,),
    in_specs=[
        pl.BlockSpec((_BLKN, _HID), lambda i: (i, 0)),
        pl.BlockSpec((2, _HID), lambda i: (0, 0)),
    ],
    out_specs=pl.BlockSpec((_BLKN, _HID), lambda i: (i, 0)),
    out_shape=jax.ShapeDtypeStruct((_N, _HID), _f32),
)


def _pool_body(h_ref, p_ref, b2_ref, wo_ref, bo_ref, out_ref, acc):
    i = pl.program_id(0)

    @pl.when(i == 0)
    def _():
        acc[...] = jnp.zeros_like(acc)

    hb = h_ref[...] * p_ref[0:1, :] + p_ref[1:2, :]
    oh = (b2_ref[...] == lax.broadcasted_iota(jnp.int32, (_BLKN, _NGRAPH), 1)
          ).astype(_f32)
    acc[...] += lax.dot_general(oh, hb, (((0,), (0,)), ((), ())),
                                preferred_element_type=_f32,
                                precision=lax.Precision.HIGHEST)

    @pl.when(i == pl.num_programs(0) - 1)
    def _():
        out_ref[...] = jnp.dot(acc[...], wo_ref[...],
                               preferred_element_type=_f32,
                               precision=lax.Precision.HIGHEST) + bo_ref[...]


_pool_call = pl.pallas_call(
    _pool_body,
    grid=(_N // _BLKN,),
    in_specs=[
        pl.BlockSpec((_BLKN, _HID), lambda i: (i, 0)),
        pl.BlockSpec((2, _HID), lambda i: (0, 0)),
        pl.BlockSpec((_BLKN, 1), lambda i: (i, 0)),
        pl.BlockSpec((_HID, 10), lambda i: (0, 0)),
        pl.BlockSpec((1, 10), lambda i: (0, 0)),
    ],
    out_specs=pl.BlockSpec((_NGRAPH, 10), lambda i: (0, 0)),
    out_shape=jax.ShapeDtypeStruct((_NGRAPH, 10), _f32),
    scratch_shapes=[pltpu.VMEM((_NGRAPH, _HID), _f32)],
)


def kernel(x, edge_index, edge_attr, batch, atom_table, bond_table, Wg, bg,
           gamma, beta, W_out, b_out):
    src = edge_index[0]
    dst = edge_index[1]

    perm = jnp.argsort(dst)
    src_p = jnp.pad(jnp.take(src, perm).astype(jnp.int32),
                    (0, _NBLKP * _K - _E)).reshape(_NBLKP, _K)
    ea_s = jnp.take(edge_attr, perm, axis=0).astype(jnp.int32)
    cnts = jnp.bincount(dst, length=_N)
    offsets = jnp.concatenate([jnp.zeros((1,), jnp.int32),
                               jnp.cumsum(cnts).astype(jnp.int32)])
    npad_off = (_NTILE - 1) * _NPT + _OFFW
    offp = jnp.concatenate(
        [offsets, jnp.full((npad_off - (_N + 1),), _E, jnp.int32)])
    offw = offp[(jnp.arange(_NTILE, dtype=jnp.int32) * _NPT)[:, None]
                + jnp.arange(_OFFW, dtype=jnp.int32)[None, :]].reshape(-1)

    h0 = _atom_call(x.astype(jnp.int32), atom_table)
    e_s = _bond_call(ea_s, bond_table)

    def edge_pass(h2):
        return _edge_call(h2, e_s, src_p, offw)[:_N]

    agg = edge_pass(h0)
    h, s1 = _upd0_call(h0, h0, agg, Wg[0], bg[0:1])
    par = _stats_call(h, s1, gamma[0:1], beta[0:1])
    for l in range(1, _NLAYER):
        h2 = _act_call(h, par)
        agg = edge_pass(h2)
        h, s1 = _upd_call(h2, h, agg, Wg[l], bg[l:l + 1])
        par = _stats_call(h, s1, gamma[l:l + 1], beta[l:l + 1])
    out = _pool_call(h, par, batch.reshape(_N, 1).astype(jnp.int32),
                     W_out, b_out.reshape(1, 10))
    return out


# speedup vs baseline: 7.8033x; 1.0014x over previous
import functools

import jax
import jax.numpy as jnp
from jax import lax
from jax.experimental import pallas as pl
from jax.experimental.pallas import tpu as pltpu
from jax.experimental.pallas import tpu_sc as plsc

_N = 10000
_E = 320000
_HID = 128
_NGRAPH = 64
_NLAYER = 7

_NTILE = 32
_NPT = 320
_NPAD = _NTILE * _NPT
_K = 128
_OFFW = 336

_BLKN = 1000
_BLKE = 1000

_f32 = jnp.float32


_SB = 8
_NBLK = _E // _K
_NBLKP = _NBLK + 2 * _SB


def _edge_body(h2_hbm, e_hbm, srcp_hbm, offw_hbm, agg_hbm,
               off_tv, idx_sb, hsrc_v, e_v, aggout, sem_d, sem_i):
    c = lax.axis_index("c")
    s = lax.axis_index("s")
    wid = s * 2 + c
    n0 = wid * _NPT
    cnt = jnp.minimum(_NPT, _N - n0)

    pltpu.sync_copy(offw_hbm.at[pl.ds(wid * _OFFW, _OFFW)],
                    off_tv.at[pl.ds(0, _OFFW)])

    def _off2(j):
        v = off_tv[pl.ds(j, 16)]
        return v[0], v[1]

    e0, _ = _off2(0)
    e1, _ = _off2(cnt)

    zeros = jnp.zeros((16,), _f32)
    has = e1 > e0

    gd0 = e0 // _K
    gend = jnp.where(has, (e1 - 1) // _K + 1, gd0)
    nbt = gend - gd0
    qb0 = gd0 // _SB
    last_q = (gend - 1) // _SB - qb0

    def _issue_idx(q):
        pltpu.async_copy(srcp_hbm.at[pl.ds((qb0 + q) * _SB, _SB)],
                         idx_sb.at[lax.rem(q, 2)], sem_i.at[lax.rem(q, 2)])

    def _wait_idx(q):
        pltpu.make_async_copy(srcp_hbm.at[pl.ds((qb0 + q) * _SB, _SB)],
                              idx_sb.at[lax.rem(q, 2)],
                              sem_i.at[lax.rem(q, 2)]).wait()

    def _chunk_slot_row(t):
        g = gd0 + t
        return lax.rem(g // _SB - qb0, 2), lax.rem(g, _SB)

    def _issue_data(t):
        slot = lax.rem(t, 2)
        cs, row = _chunk_slot_row(t)
        pltpu.async_copy(h2_hbm.at[idx_sb.at[cs, row]],
                         hsrc_v.at[slot], sem_d.at[slot])
        pltpu.async_copy(e_hbm.at[pl.ds((gd0 + t) * _K, _K)], e_v.at[slot],
                         sem_d.at[slot])

    def _wait_data(t):
        slot = lax.rem(t, 2)
        cs, row = _chunk_slot_row(t)
        pltpu.make_async_copy(h2_hbm.at[idx_sb.at[cs, row]],
                              hsrc_v.at[slot], sem_d.at[slot]).wait()
        pltpu.make_async_copy(e_hbm.at[pl.ds((gd0 + t) * _K, _K)],
                              e_v.at[slot], sem_d.at[slot]).wait()

    @pl.when(has)
    def _():
        _issue_idx(0)
        _wait_idx(0)

        @pl.when(last_q >= 1)
        def _():
            _issue_idx(1)

        _issue_data(0)

        @pl.when(nbt >= 2)
        def _():
            @pl.when(lax.rem(gd0 + 1, _SB) == 0)
            def _():
                _wait_idx(1)

                @pl.when(last_q >= 2)
                def _():
                    _issue_idx(2)

            _issue_data(1)

        _wait_data(0)

    def _node(ld, st0):
        oa, ob = _off2(ld)
        r = ob - oa

        def _edge(_, st):
            pos = st[0]
            M = list(st[1:9])
            D = list(st[9:17])
            S = list(st[17:25])
            t = pos // _K - gd0

            @pl.when(jnp.logical_and(jnp.bitwise_and(pos, _K - 1) == 0,
                                     pos > e0))
            def _():
                tn = t + 1
                cn = (gd0 + tn) // _SB - qb0

                @pl.when(jnp.logical_and(lax.rem(gd0 + tn, _SB) == 0,
                                         cn <= last_q))
                def _():
                    _wait_idx(cn)

                    @pl.when(cn + 1 <= last_q)
                    def _():
                        _issue_idx(cn + 1)

                @pl.when(tn < nbt)
                def _():
                    _issue_data(tn)

                _wait_data(t)

            slot = lax.rem(t, 2)
            i = jnp.bitwise_and(pos, _K - 1)
            for cc in range(8):
                hv = hsrc_v[slot, i, pl.ds(cc * 16, 16)]
                ev = e_v[slot, i, pl.ds(cc * 16, 16)]
                m = jnp.maximum(hv + ev, 0.0) + 1e-7
                mn = jnp.maximum(M[cc], m)
                a = jnp.exp(M[cc] - mn)
                b = jnp.exp(m - mn)
                D[cc] = D[cc] * a + b
                S[cc] = S[cc] * a + b * m
                M[cc] = mn
            return tuple([pos + 1] + M + D + S)

        st = lax.fori_loop(0, r, _edge, tuple([st0] + [zeros] * 24))
        for cc in range(8):
            aggout[ld, pl.ds(cc * 16, 16)] = st[17 + cc] / (st[9 + cc] + 1e-16)
        return st[0]

    lax.fori_loop(0, cnt, _node, e0)

    pltpu.sync_copy(aggout, agg_hbm.at[pl.ds(n0, _NPT)])


_edge_call = pl.kernel(
    _edge_body,
    out_type=jax.ShapeDtypeStruct((_NPAD, _HID), _f32),
    mesh=plsc.VectorSubcoreMesh(core_axis_name="c", subcore_axis_name="s",
                                num_cores=2, num_subcores=16),
    compiler_params=pltpu.CompilerParams(needs_layout_passes=False),
    scratch_types=[
        pltpu.VMEM((_OFFW + 16,), jnp.int32),
        pltpu.VMEM((2, _SB, _K), jnp.int32),
        pltpu.VMEM((2, _K, _HID), _f32),
        pltpu.VMEM((2, _K, _HID), _f32),
        pltpu.VMEM((_NPT, _HID), _f32),
        pltpu.SemaphoreType.DMA((2,)),
        pltpu.SemaphoreType.DMA((2,)),
    ],
)


def _enc_body(ncat, ncols, xb_ref, tab_ref, out_ref):
    xb = xb_ref[...]
    blk = xb.shape[0]
    mh = jnp.zeros((blk, ncols), _f32)
    for k in range(ncat):
        col = xb[:, k:k + 1] + jnp.int32(k * (ncols // ncat))
        mh = mh + (col == lax.broadcasted_iota(jnp.int32, (blk, ncols), 1)
                   ).astype(_f32)
    out_ref[...] = jnp.dot(mh, tab_ref[...], preferred_element_type=_f32,
                           precision=lax.Precision.HIGHEST)


def _make_enc(nrows, blk, ncat, ncols):
    return pl.pallas_call(
        functools.partial(_enc_body, ncat, ncols),
        grid=(nrows // blk,),
        in_specs=[
            pl.BlockSpec((blk, ncat), lambda i: (i, 0)),
            pl.BlockSpec((ncols, _HID), lambda i: (0, 0)),
        ],
        out_specs=pl.BlockSpec((blk, _HID), lambda i: (i, 0)),
        out_shape=jax.ShapeDtypeStruct((nrows, _HID), _f32),
    )


_atom_call = _make_enc(_N, _BLKN, 9, 288)
_bond_call = _make_enc(_E, _BLKE, 3, 24)


def _upd_body(res, h2_ref, h_ref, agg_ref, w_ref, b_ref,
              hnew_ref, s1_ref, stat):
    i = pl.program_id(0)

    @pl.when(i == 0)
    def _():
        stat[...] = jnp.zeros_like(stat)

    hn = jnp.dot(h2_ref[...] + agg_ref[...], w_ref[...],
                 preferred_element_type=_f32,
                 precision=lax.Precision.HIGHEST) + b_ref[...]
    if res:
        hn = hn + h_ref[...]
    hnew_ref[...] = hn
    stat[0:1, :] += jnp.sum(hn, axis=0, keepdims=True)

    @pl.when(i == pl.num_programs(0) - 1)
    def _():
        s1_ref[...] = stat[...]


def _make_upd(res):
    return pl.pallas_call(
        functools.partial(_upd_body, res),
        grid=(_N // _BLKN,),
        in_specs=[
            pl.BlockSpec((_BLKN, _HID), lambda i: (i, 0)),
            pl.BlockSpec((_BLKN, _HID), lambda i: (i, 0)),
            pl.BlockSpec((_BLKN, _HID), lambda i: (i, 0)),
            pl.BlockSpec((_HID, _HID), lambda i: (0, 0)),
            pl.BlockSpec((1, _HID), lambda i: (0, 0)),
        ],
        out_specs=[
            pl.BlockSpec((_BLKN, _HID), lambda i: (i, 0)),
            pl.BlockSpec((1, _HID), lambda i: (0, 0)),
        ],
        out_shape=[
            jax.ShapeDtypeStruct((_N, _HID), _f32),
            jax.ShapeDtypeStruct((1, _HID), _f32),
        ],
        scratch_shapes=[pltpu.VMEM((1, _HID), _f32)],
    )


_upd0_call = _make_upd(False)
_upd_call = _make_upd(True)


def _stats_body(h_ref, s1_ref, g_ref, par_ref, acc):
    i = pl.program_id(0)

    @pl.when(i == 0)
    def _():
        acc[...] = jnp.zeros_like(acc)

    mu = s1_ref[...] * (1.0 / _N)
    d = h_ref[...] - mu
    acc[...] += jnp.sum(d * d, axis=0, keepdims=True)

    @pl.when(i == pl.num_programs(0) - 1)
    def _():
        var = acc[...] * (1.0 / _N)
        par_ref[0:1, :] = g_ref[...] / jnp.sqrt(var + 1e-5)
        par_ref[1:2, :] = mu


_stats_call = pl.pallas_call(
    _stats_body,
    grid=(_N // _BLKN,),
    in_specs=[
        pl.BlockSpec((_BLKN, _HID), lambda i: (i, 0)),
        pl.BlockSpec((1, _HID), lambda i: (0, 0)),
        pl.BlockSpec((1, _HID), lambda i: (0, 0)),
    ],
    out_specs=pl.BlockSpec((2, _HID), lambda i: (0, 0)),
    out_shape=jax.ShapeDtypeStruct((2, _HID), _f32),
    scratch_shapes=[pltpu.VMEM((1, _HID), _f32)],
)


def _act_body(h_ref, p_ref, be_ref, out_ref):
    out_ref[...] = jnp.maximum(
        (h_ref[...] - p_ref[1:2, :]) * p_ref[0:1, :] + be_ref[...], 0.0)


_act_call = pl.pallas_call(
    _act_body,
    grid=(_N // _BLKN,),
    in_specs=[
        pl.BlockSpec((_BLKN, _HID), lambda i: (i, 0)),
        pl.BlockSpec((2, _HID), lambda i: (0, 0)),
        pl.BlockSpec((1, _HID), lambda i: (0, 0)),
    ],
    out_specs=pl.BlockSpec((_BLKN, _HID), lambda i: (i, 0)),
    out_shape=jax.ShapeDtypeStruct((_N, _HID), _f32),
)


def _pool_body(h_ref, p_ref, be_ref, b2_ref, wo_ref, bo_ref, out_ref, acc):
    i = pl.program_id(0)

    @pl.when(i == 0)
    def _():
        acc[...] = jnp.zeros_like(acc)

    hb = (h_ref[...] - p_ref[1:2, :]) * p_ref[0:1, :] + be_ref[...]
    oh = (b2_ref[...] == lax.broadcasted_iota(jnp.int32, (_BLKN, _NGRAPH), 1)
          ).astype(_f32)
    acc[...] += lax.dot_general(oh, hb, (((0,), (0,)), ((), ())),
                                preferred_element_type=_f32,
                                precision=lax.Precision.HIGHEST)

    @pl.when(i == pl.num_programs(0) - 1)
    def _():
        out_ref[...] = jnp.dot(acc[...], wo_ref[...],
                               preferred_element_type=_f32,
                               precision=lax.Precision.HIGHEST) + bo_ref[...]


_pool_call = pl.pallas_call(
    _pool_body,
    grid=(_N // _BLKN,),
    in_specs=[
        pl.BlockSpec((_BLKN, _HID), lambda i: (i, 0)),
        pl.BlockSpec((2, _HID), lambda i: (0, 0)),
        pl.BlockSpec((1, _HID), lambda i: (0, 0)),
        pl.BlockSpec((_BLKN, 1), lambda i: (i, 0)),
        pl.BlockSpec((_HID, 10), lambda i: (0, 0)),
        pl.BlockSpec((1, 10), lambda i: (0, 0)),
    ],
    out_specs=pl.BlockSpec((_NGRAPH, 10), lambda i: (0, 0)),
    out_shape=jax.ShapeDtypeStruct((_NGRAPH, 10), _f32),
    scratch_shapes=[pltpu.VMEM((_NGRAPH, _HID), _f32)],
)


def kernel(x, edge_index, edge_attr, batch, atom_table, bond_table, Wg, bg,
           gamma, beta, W_out, b_out):
    src = edge_index[0]
    dst = edge_index[1]

    perm = jnp.argsort(dst)
    src_p = jnp.pad(jnp.take(src, perm).astype(jnp.int32),
                    (0, _NBLKP * _K - _E)).reshape(_NBLKP, _K)
    ea_s = jnp.take(edge_attr, perm, axis=0).astype(jnp.int32)
    cnts = jnp.bincount(dst, length=_N)
    offsets = jnp.concatenate([jnp.zeros((1,), jnp.int32),
                               jnp.cumsum(cnts).astype(jnp.int32)])
    npad_off = (_NTILE - 1) * _NPT + _OFFW
    offp = jnp.concatenate(
        [offsets, jnp.full((npad_off - (_N + 1),), _E, jnp.int32)])
    offw = offp[(jnp.arange(_NTILE, dtype=jnp.int32) * _NPT)[:, None]
                + jnp.arange(_OFFW, dtype=jnp.int32)[None, :]].reshape(-1)

    h0 = _atom_call(x.astype(jnp.int32), atom_table)
    e_s = _bond_call(ea_s, bond_table)

    def edge_pass(h2):
        return _edge_call(h2, e_s, src_p, offw)[:_N]

    agg = edge_pass(h0)
    h, s1 = _upd0_call(h0, h0, agg, Wg[0], bg[0:1])
    par = _stats_call(h, s1, gamma[0:1])
    for l in range(1, _NLAYER):
        h2 = _act_call(h, par, beta[l - 1:l])
        agg = edge_pass(h2)
        h, s1 = _upd_call(h2, h, agg, Wg[l], bg[l:l + 1])
        par = _stats_call(h, s1, gamma[l:l + 1])
    out = _pool_call(h, par, beta[_NLAYER - 1:_NLAYER],
                     batch.reshape(_N, 1).astype(jnp.int32),
                     W_out, b_out.reshape(1, 10))
    return out
